# baseline (speedup 1.0000x reference)
import functools

import jax
import jax.numpy as jnp
from jax import lax
from jax.experimental import pallas as pl
from jax.experimental.pallas import tpu as pltpu
from jax.experimental.pallas import tpu_sc as plsc

BATCH = 16384
EMBED_K = 16
NUM_WORKERS = 32
PAIRS_PER_WORKER = BATCH // NUM_WORKERS
IDX_CHUNK = 128
NUM_CHUNKS = PAIRS_PER_WORKER // IDX_CHUNK


@functools.partial(
    pl.kernel,
    out_type=jax.ShapeDtypeStruct((BATCH,), jnp.float32),
    mesh=plsc.VectorSubcoreMesh(core_axis_name="c", subcore_axis_name="s"),
    compiler_params=pltpu.CompilerParams(use_tc_tiling_on_sc=False),
    scratch_types=[
        pltpu.VMEM((NUM_CHUNKS, IDX_CHUNK), jnp.int32),
        pltpu.VMEM((NUM_CHUNKS, IDX_CHUNK), jnp.int32),
        pltpu.VMEM((PAIRS_PER_WORKER, EMBED_K), jnp.float32),
        pltpu.VMEM((PAIRS_PER_WORKER, EMBED_K), jnp.float32),
        pltpu.VMEM((PAIRS_PER_WORKER,), jnp.float32),
        pltpu.SemaphoreType.DMA,
    ],
)
def _mf_sc_kernel(uidx_hbm, vidx_hbm, w_hbm, h_hbm, out_hbm,
                  uidx_v, vidx_v, u_v, v_v, out_v, sem):
    num_cores = 2
    wid = lax.axis_index("s") * num_cores + lax.axis_index("c")
    base = wid * PAIRS_PER_WORKER

    pltpu.sync_copy(uidx_hbm.at[wid], uidx_v)
    pltpu.sync_copy(vidx_hbm.at[wid], vidx_v)

    copies = []
    for j in range(NUM_CHUNKS):
        dst = pl.ds(j * IDX_CHUNK, IDX_CHUNK)
        copies.append(pltpu.async_copy(w_hbm.at[uidx_v.at[j]], u_v.at[dst], sem))
        copies.append(pltpu.async_copy(h_hbm.at[vidx_v.at[j]], v_v.at[dst], sem))
    for c in copies:
        c.wait()

    lane = lax.iota(jnp.int32, EMBED_K)
    folds = [lane ^ 8, lane ^ 4, lane ^ 2, lane ^ 1]
    masks = [lane % (2 * g) < g for g in (8, 4, 2, 1)]
    bitrev = [0, 8, 4, 12, 2, 10, 6, 14, 1, 9, 5, 13, 3, 11, 7, 15]

    gather_dnums = lax.GatherDimensionNumbers(
        offset_dims=(), collapsed_slice_dims=(0,), start_index_map=(0,))

    def permute(vec, idx):
        return lax.gather(vec, idx[:, None], gather_dnums, (1,),
                          mode=lax.GatherScatterMode.PROMISE_IN_BOUNDS)

    def fold(vec, level):
        return vec + permute(vec, folds[level])

    def block_body(blk, carry):
        regs = []
        for i in bitrev:
            r = blk * EMBED_K + i
            regs.append(u_v[r] * v_v[r])
        for level in range(4):
            nxt = []
            for j in range(0, len(regs), 2):
                a = fold(regs[j], level)
                b = fold(regs[j + 1], level)
                nxt.append(jnp.where(masks[level], a, b))
            regs = nxt
        acc = regs[0]
        sig = 1.0 / (1.0 + jnp.exp(-acc))
        out_v[pl.ds(blk * EMBED_K, EMBED_K)] = sig
        return carry

    lax.fori_loop(0, PAIRS_PER_WORKER // EMBED_K, block_body, 0)

    pltpu.sync_copy(out_v, out_hbm.at[pl.ds(base, PAIRS_PER_WORKER)])


TR_COLS = 2048
TR_GRID = -(-1000000 // TR_COLS)


def _transpose_body(i_ref, o_ref):
    blk = i_ref[...]
    o_ref[...] = lax.dot_general(
        blk, jnp.eye(EMBED_K, dtype=jnp.float32),
        (((0,), (0,)), ((), ())),
        preferred_element_type=jnp.float32)


def _to_row_major(wt):
    return pl.pallas_call(
        _transpose_body,
        grid=(TR_GRID,),
        in_specs=[pl.BlockSpec((EMBED_K, TR_COLS), lambda g: (0, g))],
        out_specs=pl.BlockSpec((TR_COLS, EMBED_K), lambda g: (g, 0)),
        out_shape=jax.ShapeDtypeStruct((1000000, EMBED_K), jnp.float32),
    )(wt)


def kernel(x, W, H):
    wd = _to_row_major(W.T)
    hd = _to_row_major(H.T)
    uidx = x[:, 0].astype(jnp.int32).reshape(NUM_WORKERS, NUM_CHUNKS, IDX_CHUNK)
    vidx = x[:, 1].astype(jnp.int32).reshape(NUM_WORKERS, NUM_CHUNKS, IDX_CHUNK)
    return _mf_sc_kernel(uidx, vidx, wd, hd)


# speedup vs baseline: 1.8599x; 1.8599x over previous
import functools

import jax
import jax.numpy as jnp
from jax import lax
from jax.experimental import pallas as pl
from jax.experimental.pallas import tpu as pltpu
from jax.experimental.pallas import tpu_sc as plsc

BATCH = 16384
EMBED_K = 16
NUM_WORKERS = 32
PAIRS_PER_WORKER = BATCH // NUM_WORKERS
CHUNK = 128
NUM_CHUNKS = PAIRS_PER_WORKER // CHUNK
SUPER_ROWS = 125000


@functools.partial(
    pl.kernel,
    out_type=jax.ShapeDtypeStruct((BATCH,), jnp.float32),
    mesh=plsc.VectorSubcoreMesh(core_axis_name="c", subcore_axis_name="s"),
    scratch_types=[
        pltpu.VMEM((NUM_CHUNKS, CHUNK), jnp.int32),
        pltpu.VMEM((NUM_CHUNKS, CHUNK), jnp.int32),
        pltpu.VMEM((NUM_CHUNKS, CHUNK), jnp.int32),
        pltpu.VMEM((NUM_CHUNKS, CHUNK), jnp.int32),
        pltpu.VMEM((2, CHUNK, 128), jnp.float32),
        pltpu.VMEM((2, CHUNK, 128), jnp.float32),
        pltpu.VMEM((PAIRS_PER_WORKER,), jnp.float32),
        pltpu.SemaphoreType.DMA,
    ],
)
def _mf_sc_kernel(uidx_hbm, vidx_hbm, w8_hbm, h8_hbm, out_hbm,
                  u_raw, v_raw, uj_v, vj_v, u_slab, v_slab, out_v, sem):
    num_cores = 2
    wid = lax.axis_index("s") * num_cores + lax.axis_index("c")
    base = wid * PAIRS_PER_WORKER

    pltpu.sync_copy(uidx_hbm.at[wid], u_raw)
    pltpu.sync_copy(vidx_hbm.at[wid], v_raw)

    for c in range(NUM_CHUNKS):
        for i in range(CHUNK // 16):
            sl = pl.ds(i * 16, 16)
            uj_v[c, sl] = u_raw[c, sl] >> 3
            vj_v[c, sl] = v_raw[c, sl] >> 3

    def fire(c):
        buf = c % 2
        return [
            pltpu.async_copy(w8_hbm.at[uj_v.at[c]], u_slab.at[buf], sem),
            pltpu.async_copy(h8_hbm.at[vj_v.at[c]], v_slab.at[buf], sem),
        ]

    lane = lax.iota(jnp.int32, 16)
    folds = [lane ^ 8, lane ^ 4, lane ^ 2, lane ^ 1]
    masks = [lane % (2 * g) < g for g in (8, 4, 2, 1)]
    bitrev = [0, 8, 4, 12, 2, 10, 6, 14, 1, 9, 5, 13, 3, 11, 7, 15]
    gd = lax.GatherDimensionNumbers(
        offset_dims=(), collapsed_slice_dims=(0,), start_index_map=(0,))

    def fold(vec, level):
        perm = lax.gather(vec, folds[level][:, None], gd, (1,),
                          mode=lax.GatherScatterMode.PROMISE_IN_BOUNDS)
        return vec + perm

    def compute_chunk(c):
        buf = c % 2

        def block_body(b, carry):
            uo = (u_raw[c, pl.ds(b * 16, 16)] & 7) << 4
            vo = (v_raw[c, pl.ds(b * 16, 16)] & 7) << 4
            regs = []
            for i in bitrev:
                r = b * 16 + i
                urow = u_slab[buf, r, pl.ds(uo[i], 16)]
                vrow = v_slab[buf, r, pl.ds(vo[i], 16)]
                regs.append(urow * vrow)
            for level in range(4):
                nxt = []
                for j in range(0, len(regs), 2):
                    a = fold(regs[j], level)
                    bb = fold(regs[j + 1], level)
                    nxt.append(jnp.where(masks[level], a, bb))
                regs = nxt
            sig = 1.0 / (1.0 + jnp.exp(-regs[0]))
            out_v[pl.ds(c * CHUNK + b * 16, 16)] = sig
            return carry

        lax.fori_loop(0, CHUNK // 16, block_body, 0)

    inflight = fire(0)
    for c in range(NUM_CHUNKS):
        for cp in inflight:
            cp.wait()
        nxt = fire(c + 1) if c + 1 < NUM_CHUNKS else []
        compute_chunk(c)
        inflight = nxt

    pltpu.sync_copy(out_v, out_hbm.at[pl.ds(base, PAIRS_PER_WORKER)])


def kernel(x, W, H):
    w8 = W.reshape(SUPER_ROWS, 128)
    h8 = H.reshape(SUPER_ROWS, 128)
    uidx = x[:, 0].astype(jnp.int32).reshape(NUM_WORKERS, NUM_CHUNKS, CHUNK)
    vidx = x[:, 1].astype(jnp.int32).reshape(NUM_WORKERS, NUM_CHUNKS, CHUNK)
    return _mf_sc_kernel(uidx, vidx, w8, h8)
